# Initial kernel scaffold; baseline (speedup 1.0000x reference)
import functools

import jax
import jax.numpy as jnp
from jax import lax
from jax.experimental import pallas as pl
from jax.experimental.pallas import tpu as pltpu
from jax.experimental.pallas import tpu_sc as plsc

B = 16384
D_USER = 16
D_TS = 15
N_BOUNDS_PAD = 1024
TS_FLAT_PAD = 15024

_info = plsc.get_sparse_core_info()
NC = _info.num_cores
NS = _info.num_subcores
NW = NC * NS
BPW = B // NW
NGRP = BPW // 16


def _body(uid_hbm, ts_hbm, utab_hbm, tsflat_hbm, bounds_hbm, mean_hbm, istd_hbm,
          out_hbm,
          idx_v, u_v, ts_v, bounds_v, tsflat_v, cmb_v, scal_v, sem):
    wid = lax.axis_index("s") * NC + lax.axis_index("c")
    base = wid * BPW

    pltpu.sync_copy(uid_hbm.at[pl.ds(base, BPW)], idx_v)
    gather = pltpu.async_copy(utab_hbm.at[idx_v], u_v, sem)

    pltpu.sync_copy(ts_hbm.at[pl.ds(base, BPW)], ts_v)
    pltpu.sync_copy(bounds_hbm, bounds_v)
    pltpu.sync_copy(tsflat_hbm, tsflat_v)
    pltpu.sync_copy(mean_hbm, scal_v.at[0])
    pltpu.sync_copy(istd_hbm, scal_v.at[1])

    mean = scal_v[0, :]
    istd = scal_v[1, :]
    iota16 = lax.iota(jnp.int32, (16,))

    def group(g, carry):
        ts16 = ts_v[pl.ds(g * 16, 16)]
        pos = jnp.zeros((16,), jnp.int32)
        step = N_BOUNDS_PAD // 2
        while step >= 1:
            probe = plsc.load_gather(bounds_v, [pos + (step - 1)])
            pos = jnp.where(probe <= ts16, pos + step, pos)
            step //= 2
        row_ids = g * 16 + iota16
        flat_base = pos * D_TS
        for j in range(D_TS):
            vals = plsc.load_gather(tsflat_v, [flat_base + j])
            plsc.store_scatter(cmb_v, [row_ids, jnp.full((16,), j, jnp.int32)],
                               vals)
        norm = (ts16 - mean) * istd
        plsc.store_scatter(cmb_v, [row_ids, jnp.full((16,), D_TS, jnp.int32)],
                           norm)
        return carry

    lax.fori_loop(0, NGRP, group, 0)

    gather.wait()
    pltpu.sync_copy(u_v, out_hbm.at[pl.ds(base, BPW), pl.ds(0, D_USER)])
    pltpu.sync_copy(cmb_v, out_hbm.at[pl.ds(base, BPW), pl.ds(D_USER, 16)])


_sc_call = functools.partial(
    pl.kernel,
    out_type=jax.ShapeDtypeStruct((B, 2 * D_USER), jnp.float32),
    mesh=plsc.VectorSubcoreMesh(core_axis_name="c", subcore_axis_name="s"),
    scratch_types=[
        pltpu.VMEM((BPW,), jnp.int32),
        pltpu.VMEM((BPW, D_USER), jnp.float32),
        pltpu.VMEM((BPW,), jnp.float32),
        pltpu.VMEM((N_BOUNDS_PAD,), jnp.float32),
        pltpu.VMEM((TS_FLAT_PAD,), jnp.float32),
        pltpu.VMEM((BPW, 16), jnp.float32),
        pltpu.VMEM((2, 16), jnp.float32),
        pltpu.SemaphoreType.DMA,
    ],
)(_body)


def kernel(user_id, timestamp, user_table, ts_table, bin_boundaries, ts_mean,
           ts_var):
    uid32 = user_id.astype(jnp.int32)
    bounds_pad = jnp.concatenate(
        [bin_boundaries,
         jnp.full((N_BOUNDS_PAD - bin_boundaries.shape[0],), jnp.inf,
                  jnp.float32)])
    tsflat = jnp.pad(ts_table.reshape(-1),
                     (0, TS_FLAT_PAD - ts_table.size))
    mean16 = jnp.full((16,), ts_mean, jnp.float32)
    istd16 = jnp.full((16,), lax.rsqrt(ts_var), jnp.float32)
    return _sc_call(uid32, timestamp, user_table, tsflat, bounds_pad, mean16,
                    istd16)


# speedup vs baseline: 2.2255x; 2.2255x over previous
import functools

import jax
import jax.numpy as jnp
from jax import lax
from jax.experimental import pallas as pl
from jax.experimental.pallas import tpu as pltpu
from jax.experimental.pallas import tpu_sc as plsc

B = 16384
D_USER = 16
D_TS = 15
N_BOUNDS_PAD = 1024
TS_FLAT_PAD = 15024

_info = plsc.get_sparse_core_info()
NC = _info.num_cores
NS = _info.num_subcores
NW = NC * NS
BPW = B // NW
NGRP = BPW // 16


def _body(uid_hbm, ts_hbm, utab_hbm, tsflat_hbm, bounds_hbm, mean_hbm, istd_hbm,
          out_hbm,
          idx_v, u_v, ts_v, bounds_v, tsflat_v, cmb_v, scal_v, sem):
    wid = lax.axis_index("s") * NC + lax.axis_index("c")
    base = wid * BPW

    pltpu.sync_copy(uid_hbm.at[pl.ds(base, BPW)], idx_v)
    gather = pltpu.async_copy(utab_hbm.at[idx_v], u_v, sem)

    pltpu.sync_copy(ts_hbm.at[pl.ds(base, BPW)], ts_v)
    pltpu.sync_copy(bounds_hbm, bounds_v)
    pltpu.sync_copy(tsflat_hbm, tsflat_v)
    pltpu.sync_copy(mean_hbm, scal_v.at[0])
    pltpu.sync_copy(istd_hbm, scal_v.at[1])

    mean = scal_v[0, :]
    istd = scal_v[1, :]
    iota16 = lax.iota(jnp.int32, 16)

    def group(g, carry):
        ts16 = ts_v[pl.ds(g * 16, 16)]
        pos = jnp.zeros((16,), jnp.int32)
        step = N_BOUNDS_PAD // 2
        while step >= 1:
            probe = plsc.load_gather(bounds_v, [pos + (step - 1)])
            pos = jnp.where(probe <= ts16, pos + step, pos)
            step //= 2
        row_ids = g * 16 + iota16
        flat_base = pos * D_TS
        for j in range(D_TS):
            vals = plsc.load_gather(tsflat_v, [flat_base + j])
            plsc.store_scatter(cmb_v, [row_ids, jnp.full((16,), j, jnp.int32)],
                               vals)
        norm = (ts16 - mean) * istd
        plsc.store_scatter(cmb_v, [row_ids, jnp.full((16,), D_TS, jnp.int32)],
                           norm)
        return carry

    lax.fori_loop(0, NGRP, group, 0)

    gather.wait()
    pltpu.sync_copy(u_v, out_hbm.at[pl.ds(base, BPW), pl.ds(0, D_USER)])
    pltpu.sync_copy(cmb_v, out_hbm.at[pl.ds(base, BPW), pl.ds(D_USER, 16)])


_sc_call = functools.partial(
    pl.kernel,
    out_type=jax.ShapeDtypeStruct((B, 2 * D_USER), jnp.float32),
    mesh=plsc.VectorSubcoreMesh(core_axis_name="c", subcore_axis_name="s"),
    scratch_types=[
        pltpu.VMEM((BPW,), jnp.int32),
        pltpu.VMEM((BPW, D_USER), jnp.float32),
        pltpu.VMEM((BPW,), jnp.float32),
        pltpu.VMEM((N_BOUNDS_PAD,), jnp.float32),
        pltpu.VMEM((TS_FLAT_PAD,), jnp.float32),
        pltpu.VMEM((BPW, 16), jnp.float32),
        pltpu.VMEM((2, 16), jnp.float32),
        pltpu.SemaphoreType.DMA,
    ],
    compiler_params=pltpu.CompilerParams(use_tc_tiling_on_sc=False,
                                         needs_layout_passes=False),
)(_body)


def kernel(user_id, timestamp, user_table, ts_table, bin_boundaries, ts_mean,
           ts_var):
    uid32 = user_id.astype(jnp.int32)
    bounds_pad = jnp.concatenate(
        [bin_boundaries,
         jnp.full((N_BOUNDS_PAD - bin_boundaries.shape[0],), jnp.inf,
                  jnp.float32)])
    tsflat = jnp.pad(ts_table.reshape(-1),
                     (0, TS_FLAT_PAD - ts_table.size))
    mean16 = jnp.full((16,), ts_mean, jnp.float32)
    istd16 = jnp.full((16,), lax.rsqrt(ts_var), jnp.float32)
    return _sc_call(uid32, timestamp, user_table, tsflat, bounds_pad, mean16,
                    istd16)
